# baseline (speedup 1.0000x reference)
import functools

import jax
import jax.numpy as jnp
from jax import lax
from jax.experimental import pallas as pl
from jax.experimental.pallas import tpu as pltpu
from jax.experimental.pallas import tpu_sc as plsc

_NC = 2
_NS = 16
_NW = _NC * _NS
_LANES = 16


def _aggregate_sc(idx_flat, x, n_pad, K, D, C, T, npw):
    mesh = plsc.VectorSubcoreMesh(core_axis_name="c", subcore_axis_name="s")
    G = C * K

    @functools.partial(
        pl.kernel,
        out_type=jax.ShapeDtypeStruct((n_pad, D), jnp.float32),
        mesh=mesh,
        scratch_types=[
            pltpu.VMEM((npw * K,), jnp.int32),
            pltpu.VMEM((2, G, D), jnp.float32),
            pltpu.VMEM((2, C, D), jnp.float32),
            pltpu.SemaphoreType.DMA,
            pltpu.SemaphoreType.DMA,
            pltpu.SemaphoreType.DMA,
            pltpu.SemaphoreType.DMA,
        ],
    )
    def aggr_kernel(idx_hbm, x_hbm, out_hbm, idx_v, rows_v, out_v,
                    g0, g1, o0, o1):
        wid = lax.axis_index("s") * _NC + lax.axis_index("c")
        gsem = (g0, g1)
        osem = (o0, o1)

        def gather_desc(t, buf):
            return pltpu.make_async_copy(
                x_hbm.at[idx_v.at[pl.ds(t * G, G)]], rows_v.at[buf],
                gsem[buf])

        def out_desc(t, buf):
            node_base = wid * npw + t * C
            return pltpu.make_async_copy(
                out_v.at[buf], out_hbm.at[pl.ds(node_base, C)], osem[buf])

        pltpu.sync_copy(idx_hbm.at[pl.ds(wid * npw * K, npw * K)], idx_v)
        gather_desc(0, 0).start()
        gather_desc(1, 1).start()

        @pl.loop(0, T // 2)
        def _(i):
            for buf in range(2):
                t = i * 2 + buf
                gather_desc(t, buf).wait()

                @pl.when(i > 0)
                def _():
                    out_desc(t, buf).wait()

                for n in range(C):
                    base = n * K
                    accs = tuple(
                        rows_v[buf, base, pl.ds(d * _LANES, _LANES)]
                        for d in range(D // _LANES)
                    )

                    def body(k, a, base=base):
                        return tuple(
                            v + rows_v[buf, base + k, pl.ds(d * _LANES, _LANES)]
                            for d, v in enumerate(a)
                        )

                    accs = lax.fori_loop(1, K, body, accs)
                    for d, v in enumerate(accs):
                        out_v[buf, n, pl.ds(d * _LANES, _LANES)] = v

                out_desc(t, buf).start()

                @pl.when(t + 2 < T)
                def _():
                    gather_desc(t + 2, buf).start()

        out_desc(T - 2, 0).wait()
        out_desc(T - 1, 1).wait()

    return aggr_kernel(idx_flat, x)


def _combine_tc(x, aggr, w1t, w2t, bias, N, D):
    BLK = 1000
    grid = (N // BLK,)

    def body(x_ref, a_ref, w1_ref, w2_ref, b_ref, o_ref):
        acc = jnp.dot(x_ref[...], w1_ref[...],
                      preferred_element_type=jnp.float32,
                      precision=lax.Precision.HIGHEST)
        acc += jnp.dot(a_ref[...], w2_ref[...],
                       preferred_element_type=jnp.float32,
                       precision=lax.Precision.HIGHEST)
        o_ref[...] = jnp.maximum(acc + b_ref[...], 0.0)

    return pl.pallas_call(
        body,
        grid=grid,
        in_specs=[
            pl.BlockSpec((BLK, D), lambda i: (i, 0)),
            pl.BlockSpec((BLK, D), lambda i: (i, 0)),
            pl.BlockSpec((D, D), lambda i: (0, 0)),
            pl.BlockSpec((D, D), lambda i: (0, 0)),
            pl.BlockSpec((1, D), lambda i: (0, 0)),
        ],
        out_specs=pl.BlockSpec((BLK, D), lambda i: (i, 0)),
        out_shape=jax.ShapeDtypeStruct((N, D), jnp.float32),
    )(x, aggr, w1t, w2t, bias)


def kernel(neighbours, shape_features, W1, b1, W2, b2):
    N, K = neighbours.shape
    D = shape_features.shape[1]

    C = 128 // K
    npw = -(-N // _NW)
    npw = -(-npw // (2 * C)) * (2 * C)
    T = npw // C
    n_pad = _NW * npw

    nbr_pad = jnp.pad(neighbours, ((0, n_pad - N), (0, 0)))
    idx_flat = nbr_pad.reshape(-1).astype(jnp.int32)

    aggr = _aggregate_sc(idx_flat, shape_features, n_pad, K, D, C, T, npw)

    bias = (b1 + b2).reshape(1, D)
    return _combine_tc(shape_features, aggr[:N], W1.T, W2.T, bias, N, D)


# speedup vs baseline: 4.4954x; 4.4954x over previous
import functools

import jax
import jax.numpy as jnp
from jax import lax
from jax.experimental import pallas as pl
from jax.experimental.pallas import tpu as pltpu
from jax.experimental.pallas import tpu_sc as plsc

_NC = 2
_NS = 16
_NW = _NC * _NS
_LANES = 16


def _aggregate_sc(idx_flat, x, n_pad, K, D, C, T, npw):
    mesh = plsc.VectorSubcoreMesh(core_axis_name="c", subcore_axis_name="s")
    G = C * K

    @functools.partial(
        pl.kernel,
        out_type=jax.ShapeDtypeStruct((n_pad, D), jnp.float32),
        mesh=mesh,
        scratch_types=[
            pltpu.VMEM((npw * K,), jnp.int32),
            pltpu.VMEM((2, G, D), jnp.float32),
            pltpu.VMEM((2, C, D), jnp.float32),
            pltpu.SemaphoreType.DMA,
            pltpu.SemaphoreType.DMA,
            pltpu.SemaphoreType.DMA,
            pltpu.SemaphoreType.DMA,
        ],
    )
    def aggr_kernel(idx_hbm, x_hbm, out_hbm, idx_v, rows_v, out_v,
                    g0, g1, o0, o1):
        wid = lax.axis_index("s") * _NC + lax.axis_index("c")
        gsem = (g0, g1)
        osem = (o0, o1)

        def gather_desc(t, buf):
            return pltpu.make_async_copy(
                x_hbm.at[idx_v.at[pl.ds(t * G, G)]], rows_v.at[buf],
                gsem[buf])

        def out_desc(t, buf):
            node_base = wid * npw + t * C
            return pltpu.make_async_copy(
                out_v.at[buf], out_hbm.at[pl.ds(node_base, C)], osem[buf])

        pltpu.sync_copy(idx_hbm.at[pl.ds(wid * npw * K, npw * K)], idx_v)
        gather_desc(0, 0).start()
        gather_desc(1, 1).start()

        @pl.loop(0, T // 2)
        def _(i):
            for buf in range(2):
                t = i * 2 + buf
                gather_desc(t, buf).wait()

                @pl.when(i > 0)
                def _():
                    out_desc(t, buf).wait()

                for n in range(C):
                    base = n * K
                    accs = tuple(
                        rows_v[buf, base, pl.ds(d * _LANES, _LANES)]
                        for d in range(D // _LANES)
                    )

                    def body(k, a, base=base):
                        return tuple(
                            v + rows_v[buf, base + k, pl.ds(d * _LANES, _LANES)]
                            for d, v in enumerate(a)
                        )

                    accs = lax.fori_loop(1, K, body, accs)
                    for d, v in enumerate(accs):
                        out_v[buf, n, pl.ds(d * _LANES, _LANES)] = v

                out_desc(t, buf).start()

                @pl.when(t + 2 < T)
                def _():
                    gather_desc(t + 2, buf).start()

        out_desc(T - 2, 0).wait()
        out_desc(T - 1, 1).wait()

    return aggr_kernel(idx_flat, x)


def _combine_tc(x, aggr, w1t, w2t, bias, N, D):
    BLK = 1000
    grid = (N // BLK,)

    def body(x_ref, a_ref, w1_ref, w2_ref, b_ref, o_ref):
        acc = jnp.dot(x_ref[...], w1_ref[...],
                      preferred_element_type=jnp.float32,
                      precision=lax.Precision.HIGHEST)
        acc += jnp.dot(a_ref[...], w2_ref[...],
                       preferred_element_type=jnp.float32,
                       precision=lax.Precision.HIGHEST)
        o_ref[...] = jnp.maximum(acc + b_ref[...], 0.0)

    return pl.pallas_call(
        body,
        grid=grid,
        in_specs=[
            pl.BlockSpec((BLK, D), lambda i: (i, 0)),
            pl.BlockSpec((BLK, D), lambda i: (i, 0)),
            pl.BlockSpec((D, D), lambda i: (0, 0)),
            pl.BlockSpec((D, D), lambda i: (0, 0)),
            pl.BlockSpec((1, D), lambda i: (0, 0)),
        ],
        out_specs=pl.BlockSpec((BLK, D), lambda i: (i, 0)),
        out_shape=jax.ShapeDtypeStruct((N, D), jnp.float32),
    )(x, aggr, w1t, w2t, bias)


def kernel(neighbours, shape_features, W1, b1, W2, b2):
    N, K = neighbours.shape
    D = shape_features.shape[1]

    C = 128 // K
    npw = -(-N // _NW)
    npw = -(-npw // (2 * C)) * (2 * C)
    T = npw // C
    n_pad = _NW * npw

    pad_rows = n_pad - N
    pad_idx = (jnp.arange(pad_rows * K, dtype=jnp.int32) % N).reshape(pad_rows, K)
    nbr_pad = jnp.concatenate([neighbours.astype(jnp.int32), pad_idx], axis=0)
    idx_flat = nbr_pad.reshape(-1)

    aggr = _aggregate_sc(idx_flat, shape_features, n_pad, K, D, C, T, npw)

    bias = (b1 + b2).reshape(1, D)
    return _combine_tc(shape_features, aggr[:N], W1.T, W2.T, bias, N, D)


# speedup vs baseline: 4.9432x; 1.0996x over previous
import functools

import jax
import jax.numpy as jnp
from jax import lax
from jax.experimental import pallas as pl
from jax.experimental.pallas import tpu as pltpu
from jax.experimental.pallas import tpu_sc as plsc

_NC = 2
_NS = 16
_NW = _NC * _NS
_LANES = 16


def _aggregate_sc(idx_flat, x, N, K, D, C, npw):
    mesh = plsc.VectorSubcoreMesh(core_axis_name="c", subcore_axis_name="s")
    G = C * K

    @functools.partial(
        pl.kernel,
        out_type=jax.ShapeDtypeStruct((N, D), jnp.float32),
        mesh=mesh,
        scratch_types=[
            pltpu.VMEM((npw * K,), jnp.int32),
            pltpu.VMEM((2, G, D), jnp.float32),
            pltpu.VMEM((2, C, D), jnp.float32),
            pltpu.SemaphoreType.DMA,
            pltpu.SemaphoreType.DMA,
            pltpu.SemaphoreType.DMA,
            pltpu.SemaphoreType.DMA,
        ],
    )
    def aggr_kernel(idx_hbm, x_hbm, out_hbm, idx_v, rows_v, out_v,
                    g0, g1, o0, o1):
        wid = lax.axis_index("s") * _NC + lax.axis_index("c")
        gsem = (g0, g1)
        osem = (o0, o1)

        node_start = wid * npw
        cnt = jnp.minimum(N - node_start, npw)
        T = cnt // C
        copy_start = jnp.minimum(node_start, N - npw) * K
        off = node_start * K - copy_start

        def gather_desc(t, buf):
            return pltpu.make_async_copy(
                x_hbm.at[idx_v.at[pl.ds(off + t * G, G)]], rows_v.at[buf],
                gsem[buf])

        def out_desc(t, buf):
            return pltpu.make_async_copy(
                out_v.at[buf], out_hbm.at[pl.ds(node_start + t * C, C)],
                osem[buf])

        pltpu.sync_copy(idx_hbm.at[pl.ds(copy_start, npw * K)], idx_v)
        gather_desc(0, 0).start()
        gather_desc(1, 1).start()

        @pl.loop(0, T // 2)
        def _(i):
            for buf in range(2):
                t = i * 2 + buf
                gather_desc(t, buf).wait()

                @pl.when(i > 0)
                def _():
                    out_desc(t, buf).wait()

                for n in range(C):
                    base = n * K
                    accs = tuple(
                        rows_v[buf, base, pl.ds(d * _LANES, _LANES)]
                        for d in range(D // _LANES)
                    )

                    def body(k, a, base=base):
                        return tuple(
                            v + rows_v[buf, base + k, pl.ds(d * _LANES, _LANES)]
                            for d, v in enumerate(a)
                        )

                    accs = lax.fori_loop(1, K, body, accs)
                    for d, v in enumerate(accs):
                        out_v[buf, n, pl.ds(d * _LANES, _LANES)] = v

                out_desc(t, buf).start()

                @pl.when(t + 2 < T)
                def _():
                    gather_desc(t + 2, buf).start()

        out_desc(T - 2, 0).wait()
        out_desc(T - 1, 1).wait()

    return aggr_kernel(idx_flat, x)


def _matmul_bias_tc(x, wt, bias, N, D):
    BLK = 1000

    def body(x_ref, w_ref, b_ref, o_ref):
        o_ref[...] = jnp.dot(
            x_ref[...], w_ref[...], preferred_element_type=jnp.float32,
            precision=lax.Precision.HIGHEST) + b_ref[...]

    return pl.pallas_call(
        body,
        grid=(N // BLK,),
        in_specs=[
            pl.BlockSpec((BLK, D), lambda i: (i, 0)),
            pl.BlockSpec((D, D), lambda i: (0, 0)),
            pl.BlockSpec((1, D), lambda i: (0, 0)),
        ],
        out_specs=pl.BlockSpec((BLK, D), lambda i: (i, 0)),
        out_shape=jax.ShapeDtypeStruct((N, D), jnp.float32),
    )(x, wt, bias)


def _combine_tc(out1, aggr, w2t, N, D):
    BLK = 1000

    def body(o1_ref, a_ref, w_ref, o_ref):
        acc = jnp.dot(a_ref[...], w_ref[...],
                      preferred_element_type=jnp.float32,
                      precision=lax.Precision.HIGHEST)
        o_ref[...] = jnp.maximum(acc + o1_ref[...], 0.0)

    return pl.pallas_call(
        body,
        grid=(N // BLK,),
        in_specs=[
            pl.BlockSpec((BLK, D), lambda i: (i, 0)),
            pl.BlockSpec((BLK, D), lambda i: (i, 0)),
            pl.BlockSpec((D, D), lambda i: (0, 0)),
        ],
        out_specs=pl.BlockSpec((BLK, D), lambda i: (i, 0)),
        out_shape=jax.ShapeDtypeStruct((N, D), jnp.float32),
    )(out1, aggr, w2t)


def kernel(neighbours, shape_features, W1, b1, W2, b2):
    N, K = neighbours.shape
    D = shape_features.shape[1]

    C = 128 // K
    npw = -(-N // _NW)
    npw = -(-npw // (2 * C)) * (2 * C)

    idx_flat = neighbours.reshape(-1).astype(jnp.int32)
    aggr = _aggregate_sc(idx_flat, shape_features, N, K, D, C, npw)

    bias = (b1 + b2).reshape(1, D)
    out1 = _matmul_bias_tc(shape_features, W1.T, bias, N, D)
    return _combine_tc(out1, aggr, W2.T, N, D)
